# speedup vs baseline: 5.4362x; 5.4362x over previous
import functools

import jax
import jax.numpy as jnp
from jax.experimental import pallas as pl
from jax.experimental.pallas import tpu as pltpu

T = 256
N = 16
D = 512
H = 512
TS = 32
LANE = 128


def _gru_layer_kernel(x_ref, wih_ref, whh_ref, bih_ref, bhh_ref, m_ref,
                      h0_ref, out_ref, h_ref, gi_ref, *, ts, n, h_dim):
    c = pl.program_id(0)

    @pl.when(c == 0)
    def _():
        h_ref[...] = h0_ref[...]

    xc = x_ref[...].reshape(ts * n, x_ref.shape[2])
    gi_ref[...] = (
        jnp.dot(xc, wih_ref[...], preferred_element_type=jnp.float32)
        + bih_ref[0:1, :]
    )

    def step(i, h):
        m = m_ref[pl.ds(i, 1), :, :].reshape(n, LANE)[:, 0:1]
        h = h * m
        gh = (
            jnp.dot(h, whh_ref[...], preferred_element_type=jnp.float32)
            + bhh_ref[0:1, :]
        )
        gi = gi_ref[pl.ds(i * n, n), :]
        r = jax.nn.sigmoid(gi[:, :h_dim] + gh[:, :h_dim])
        z = jax.nn.sigmoid(gi[:, h_dim:2 * h_dim] + gh[:, h_dim:2 * h_dim])
        nn_ = jnp.tanh(gi[:, 2 * h_dim:] + r * gh[:, 2 * h_dim:])
        h = (1.0 - z) * nn_ + z * h
        out_ref[pl.ds(i, 1)] = h.reshape(1, n, h_dim)
        return h

    h_ref[...] = jax.lax.fori_loop(0, ts, step, h_ref[...])


def _gru_layer(xs, h0, masks_b, w_ih_t, w_hh_t, b_ih, b_hh):
    d_in = xs.shape[2]
    body = functools.partial(_gru_layer_kernel, ts=TS, n=N, h_dim=H)
    return pl.pallas_call(
        body,
        grid=(T // TS,),
        in_specs=[
            pl.BlockSpec((TS, N, d_in), lambda c: (c, 0, 0)),
            pl.BlockSpec((d_in, 3 * H), lambda c: (0, 0)),
            pl.BlockSpec((H, 3 * H), lambda c: (0, 0)),
            pl.BlockSpec((8, 3 * H), lambda c: (0, 0)),
            pl.BlockSpec((8, 3 * H), lambda c: (0, 0)),
            pl.BlockSpec((TS, N, LANE), lambda c: (c, 0, 0)),
            pl.BlockSpec((N, H), lambda c: (0, 0)),
        ],
        out_specs=pl.BlockSpec((TS, N, H), lambda c: (c, 0, 0)),
        out_shape=jax.ShapeDtypeStruct((T, N, H), jnp.float32),
        scratch_shapes=[
            pltpu.VMEM((N, H), jnp.float32),
            pltpu.VMEM((TS * N, 3 * H), jnp.float32),
        ],
        compiler_params=pltpu.CompilerParams(
            dimension_semantics=("arbitrary",),
        ),
    )(xs, w_ih_t, w_hh_t, b_ih, b_hh, masks_b, h0)


def kernel(x, hidden_states, masks, W_ih0, W_hh0, b_ih0, b_hh0,
           W_ih1, W_hh1, b_ih1, b_hh1):
    xs = x.reshape(T, N, D)
    masks_b = jnp.broadcast_to(
        masks.astype(jnp.float32).reshape(T, N, 1), (T, N, LANE))

    def prep_b(b):
        return jnp.broadcast_to(b.reshape(1, 3 * H), (8, 3 * H))

    out0 = _gru_layer(xs, hidden_states[0], masks_b,
                      W_ih0.T, W_hh0.T, prep_b(b_ih0), prep_b(b_hh0))
    out1 = _gru_layer(out0, hidden_states[1], masks_b,
                      W_ih1.T, W_hh1.T, prep_b(b_ih1), prep_b(b_hh1))

    h_n = jnp.stack([out0[T - 1], out1[T - 1]], axis=0)
    return out1.reshape(T * N, H), h_n


# speedup vs baseline: 5.4677x; 1.0058x over previous
import functools

import jax
import jax.numpy as jnp
from jax.experimental import pallas as pl
from jax.experimental.pallas import tpu as pltpu

T = 256
N = 16
D = 512
H = 512
TS = 32
LANE = 128


def _gru_steps(gi_ref, whh_ref, bhh_ref, m_ref, h, emit, *, ts, n, h_dim):
    def step(i, h):
        m = m_ref[pl.ds(i, 1), :, :].reshape(n, LANE)[:, 0:1]
        h = h * m
        gh = (
            jnp.dot(h, whh_ref[...], preferred_element_type=jnp.float32)
            + bhh_ref[0:1, :]
        )
        gi = gi_ref[pl.ds(i * n, n), :]
        r = jax.nn.sigmoid(gi[:, :h_dim] + gh[:, :h_dim])
        z = jax.nn.sigmoid(gi[:, h_dim:2 * h_dim] + gh[:, h_dim:2 * h_dim])
        cand = jnp.tanh(gi[:, 2 * h_dim:] + r * gh[:, 2 * h_dim:])
        h = (1.0 - z) * cand + z * h
        emit(i, h)
        return h

    return jax.lax.fori_loop(0, ts, step, h)


def _gru2_kernel(x_ref, wih0_ref, whh0_ref, bih0_ref, bhh0_ref,
                 wih1_ref, whh1_ref, bih1_ref, bhh1_ref, m_ref, hinit_ref,
                 out_ref, hn_ref, h0_ref, h1_ref, gi0_ref, mid_ref, gi1_ref,
                 *, ts, n, h_dim, nchunks):
    c = pl.program_id(0)

    @pl.when(c == 0)
    def _():
        h0_ref[...] = hinit_ref[0]
        h1_ref[...] = hinit_ref[1]

    xc = x_ref[...].reshape(ts * n, x_ref.shape[2])
    gi0_ref[...] = (
        jnp.dot(xc, wih0_ref[...], preferred_element_type=jnp.float32)
        + bih0_ref[0:1, :]
    )

    def emit0(i, h):
        mid_ref[pl.ds(i * n, n), :] = h

    h0_ref[...] = _gru_steps(gi0_ref, whh0_ref, bhh0_ref, m_ref, h0_ref[...],
                             emit0, ts=ts, n=n, h_dim=h_dim)

    gi1_ref[...] = (
        jnp.dot(mid_ref[...], wih1_ref[...], preferred_element_type=jnp.float32)
        + bih1_ref[0:1, :]
    )

    def emit1(i, h):
        out_ref[pl.ds(i, 1)] = h.reshape(1, n, h_dim)

    h1_ref[...] = _gru_steps(gi1_ref, whh1_ref, bhh1_ref, m_ref, h1_ref[...],
                             emit1, ts=ts, n=n, h_dim=h_dim)

    @pl.when(c == nchunks - 1)
    def _():
        hn_ref[0] = h0_ref[...]
        hn_ref[1] = h1_ref[...]


def kernel(x, hidden_states, masks, W_ih0, W_hh0, b_ih0, b_hh0,
           W_ih1, W_hh1, b_ih1, b_hh1):
    xs = x.reshape(T, N, D)
    masks_b = jnp.broadcast_to(
        masks.astype(jnp.float32).reshape(T, N, 1), (T, N, LANE))

    def prep_b(b):
        return jnp.broadcast_to(b.reshape(1, 3 * H), (8, 3 * H))

    nchunks = T // TS
    body = functools.partial(_gru2_kernel, ts=TS, n=N, h_dim=H,
                             nchunks=nchunks)
    full = lambda shape: pl.BlockSpec(shape, lambda c: (0,) * len(shape))
    out1, h_n = pl.pallas_call(
        body,
        grid=(nchunks,),
        in_specs=[
            pl.BlockSpec((TS, N, D), lambda c: (c, 0, 0)),
            full((D, 3 * H)),
            full((H, 3 * H)),
            full((8, 3 * H)),
            full((8, 3 * H)),
            full((H, 3 * H)),
            full((H, 3 * H)),
            full((8, 3 * H)),
            full((8, 3 * H)),
            pl.BlockSpec((TS, N, LANE), lambda c: (c, 0, 0)),
            full((2, N, H)),
        ],
        out_specs=[
            pl.BlockSpec((TS, N, H), lambda c: (c, 0, 0)),
            full((2, N, H)),
        ],
        out_shape=[
            jax.ShapeDtypeStruct((T, N, H), jnp.float32),
            jax.ShapeDtypeStruct((2, N, H), jnp.float32),
        ],
        scratch_shapes=[
            pltpu.VMEM((N, H), jnp.float32),
            pltpu.VMEM((N, H), jnp.float32),
            pltpu.VMEM((TS * N, 3 * H), jnp.float32),
            pltpu.VMEM((TS * N, H), jnp.float32),
            pltpu.VMEM((TS * N, 3 * H), jnp.float32),
        ],
        compiler_params=pltpu.CompilerParams(
            dimension_semantics=("arbitrary",),
        ),
    )(xs, W_ih0.T, W_hh0.T, prep_b(b_ih0), prep_b(b_hh0),
      W_ih1.T, W_hh1.T, prep_b(b_ih1), prep_b(b_hh1),
      masks_b, hidden_states)

    return out1.reshape(T * N, H), h_n


# speedup vs baseline: 5.4824x; 1.0027x over previous
import functools

import jax
import jax.numpy as jnp
from jax.experimental import pallas as pl
from jax.experimental.pallas import tpu as pltpu

T = 256
N = 16
D = 512
H = 512
TS = 32
LANE = 128


def _gru_steps(gi_ref, whh_ref, bhh_ref, m_ref, h, emit, *, ts, n, h_dim):
    def step(i, h):
        m = m_ref[pl.ds(i, 1), :, :].reshape(n, LANE)[:, 0:1]
        h = h * m
        gh = (
            jnp.dot(h.astype(jnp.bfloat16), whh_ref[...],
                    preferred_element_type=jnp.float32)
            + bhh_ref[0:1, :]
        )
        gi = gi_ref[pl.ds(i * n, n), :]
        r = jax.nn.sigmoid(gi[:, :h_dim] + gh[:, :h_dim])
        z = jax.nn.sigmoid(gi[:, h_dim:2 * h_dim] + gh[:, h_dim:2 * h_dim])
        cand = jnp.tanh(gi[:, 2 * h_dim:] + r * gh[:, 2 * h_dim:])
        h = (1.0 - z) * cand + z * h
        emit(i, h)
        return h

    return jax.lax.fori_loop(0, ts, step, h)


def _gru2_kernel(x_ref, wih0_ref, whh0_ref, bih0_ref, bhh0_ref,
                 wih1_ref, whh1_ref, bih1_ref, bhh1_ref, m_ref, hinit_ref,
                 out_ref, hn_ref, h0_ref, h1_ref, gi0_ref, mid_ref, gi1_ref,
                 *, ts, n, h_dim, nchunks):
    c = pl.program_id(0)

    @pl.when(c == 0)
    def _():
        h0_ref[...] = hinit_ref[0]
        h1_ref[...] = hinit_ref[1]

    xc = x_ref[...].reshape(ts * n, x_ref.shape[2])
    gi0_ref[...] = (
        jnp.dot(xc, wih0_ref[...], preferred_element_type=jnp.float32)
        + bih0_ref[0:1, :]
    )

    def emit0(i, h):
        mid_ref[pl.ds(i * n, n), :] = h.astype(jnp.bfloat16)

    h0_ref[...] = _gru_steps(gi0_ref, whh0_ref, bhh0_ref, m_ref, h0_ref[...],
                             emit0, ts=ts, n=n, h_dim=h_dim)

    gi1_ref[...] = (
        jnp.dot(mid_ref[...], wih1_ref[...], preferred_element_type=jnp.float32)
        + bih1_ref[0:1, :]
    )

    def emit1(i, h):
        out_ref[pl.ds(i, 1)] = h.reshape(1, n, h_dim)

    h1_ref[...] = _gru_steps(gi1_ref, whh1_ref, bhh1_ref, m_ref, h1_ref[...],
                             emit1, ts=ts, n=n, h_dim=h_dim)

    @pl.when(c == nchunks - 1)
    def _():
        hn_ref[0] = h0_ref[...]
        hn_ref[1] = h1_ref[...]


def kernel(x, hidden_states, masks, W_ih0, W_hh0, b_ih0, b_hh0,
           W_ih1, W_hh1, b_ih1, b_hh1):
    xs = x.reshape(T, N, D).astype(jnp.bfloat16)
    masks_b = jnp.broadcast_to(
        masks.astype(jnp.float32).reshape(T, N, 1), (T, N, LANE))

    def prep_b(b):
        return jnp.broadcast_to(b.reshape(1, 3 * H), (8, 3 * H))

    nchunks = T // TS
    body = functools.partial(_gru2_kernel, ts=TS, n=N, h_dim=H,
                             nchunks=nchunks)
    full = lambda shape: pl.BlockSpec(shape, lambda c: (0,) * len(shape))
    out1, h_n = pl.pallas_call(
        body,
        grid=(nchunks,),
        in_specs=[
            pl.BlockSpec((TS, N, D), lambda c: (c, 0, 0)),
            full((D, 3 * H)),
            full((H, 3 * H)),
            full((8, 3 * H)),
            full((8, 3 * H)),
            full((H, 3 * H)),
            full((H, 3 * H)),
            full((8, 3 * H)),
            full((8, 3 * H)),
            pl.BlockSpec((TS, N, LANE), lambda c: (c, 0, 0)),
            full((2, N, H)),
        ],
        out_specs=[
            pl.BlockSpec((TS, N, H), lambda c: (c, 0, 0)),
            full((2, N, H)),
        ],
        out_shape=[
            jax.ShapeDtypeStruct((T, N, H), jnp.float32),
            jax.ShapeDtypeStruct((2, N, H), jnp.float32),
        ],
        scratch_shapes=[
            pltpu.VMEM((N, H), jnp.float32),
            pltpu.VMEM((N, H), jnp.float32),
            pltpu.VMEM((TS * N, 3 * H), jnp.float32),
            pltpu.VMEM((TS * N, H), jnp.bfloat16),
            pltpu.VMEM((TS * N, 3 * H), jnp.float32),
        ],
        compiler_params=pltpu.CompilerParams(
            dimension_semantics=("arbitrary",),
        ),
    )(xs, W_ih0.T.astype(jnp.bfloat16), W_hh0.T.astype(jnp.bfloat16),
      prep_b(b_ih0), prep_b(b_hh0),
      W_ih1.T.astype(jnp.bfloat16), W_hh1.T.astype(jnp.bfloat16),
      prep_b(b_ih1), prep_b(b_hh1),
      masks_b, hidden_states)

    return out1.reshape(T * N, H), h_n


# speedup vs baseline: 6.0719x; 1.1075x over previous
import functools

import jax
import jax.numpy as jnp
from jax.experimental import pallas as pl
from jax.experimental.pallas import tpu as pltpu

T = 256
N = 16
D = 512
H = 512
TS = 32
LANE = 128


def _gru_gate(h, m, gh_unbiased, bhh, gi, h_dim):
    gh = gh_unbiased + bhh
    r = jax.nn.sigmoid(gi[:, :h_dim] + gh[:, :h_dim])
    z = jax.nn.sigmoid(gi[:, h_dim:2 * h_dim] + gh[:, h_dim:2 * h_dim])
    cand = jnp.tanh(gi[:, 2 * h_dim:] + r * gh[:, 2 * h_dim:])
    return (1.0 - z) * cand + z * (h * m)


def _gru2_kernel(x_ref, wih0_ref, whh0_ref, bih0_ref, bhh0_ref,
                 wih1_ref, whh1_ref, bih1_ref, bhh1_ref, m0_ref, m1_ref,
                 hinit_ref, out_ref, hn_ref,
                 h0_ref, h1_ref, gi0_ref, mid_ref, gi1_ref,
                 *, ts, n, h_dim, nchunks):
    c = pl.program_id(0)

    @pl.when(c == 0)
    def _():
        h0_ref[...] = hinit_ref[0]
        h1_ref[...] = hinit_ref[1]
        gi1_ref[...] = jnp.zeros_like(gi1_ref)

    @pl.when(c < nchunks)
    def _():
        xc = x_ref[...].reshape(ts * n, x_ref.shape[2])
        gi0_ref[...] = (
            jnp.dot(xc, wih0_ref[...], preferred_element_type=jnp.float32)
            + bih0_ref[0:1, :]
        )

    l0_on = c < nchunks
    l1_on = c > 0

    def step(i, carry):
        h0, h1 = carry
        m0 = m0_ref[pl.ds(i, 1), :, :].reshape(n, LANE)[:, 0:1]
        m1 = m1_ref[pl.ds(i, 1), :, :].reshape(n, LANE)[:, 0:1]
        gh0 = jnp.dot((h0 * m0).astype(jnp.bfloat16), whh0_ref[...],
                      preferred_element_type=jnp.float32)
        gh1 = jnp.dot((h1 * m1).astype(jnp.bfloat16), whh1_ref[...],
                      preferred_element_type=jnp.float32)
        h0n = _gru_gate(h0, m0, gh0, bhh0_ref[0:1, :],
                        gi0_ref[pl.ds(i * n, n), :], h_dim)
        h1n = _gru_gate(h1, m1, gh1, bhh1_ref[0:1, :],
                        gi1_ref[pl.ds(i * n, n), :], h_dim)
        mid_ref[pl.ds(i * n, n), :] = h0n.astype(jnp.bfloat16)
        out_ref[pl.ds(i, 1)] = h1n.reshape(1, n, h_dim)
        h0 = jnp.where(l0_on, h0n, h0)
        h1 = jnp.where(l1_on, h1n, h1)
        return (h0, h1)

    h0, h1 = jax.lax.fori_loop(0, ts, step, (h0_ref[...], h1_ref[...]))
    h0_ref[...] = h0
    h1_ref[...] = h1

    @pl.when(c == nchunks)
    def _():
        hn_ref[0] = h0_ref[...]
        hn_ref[1] = h1_ref[...]

    @pl.when(c < nchunks)
    def _():
        gi1_ref[...] = (
            jnp.dot(mid_ref[...], wih1_ref[...],
                    preferred_element_type=jnp.float32)
            + bih1_ref[0:1, :]
        )


def kernel(x, hidden_states, masks, W_ih0, W_hh0, b_ih0, b_hh0,
           W_ih1, W_hh1, b_ih1, b_hh1):
    xs = x.reshape(T, N, D).astype(jnp.bfloat16)
    masks_b = jnp.broadcast_to(
        masks.astype(jnp.float32).reshape(T, N, 1), (T, N, LANE))

    def prep_b(b):
        return jnp.broadcast_to(b.reshape(1, 3 * H), (8, 3 * H))

    nchunks = T // TS
    last = nchunks - 1
    body = functools.partial(_gru2_kernel, ts=TS, n=N, h_dim=H,
                             nchunks=nchunks)
    full = lambda shape: pl.BlockSpec(shape, lambda c: (0,) * len(shape))
    out1, h_n = pl.pallas_call(
        body,
        grid=(nchunks + 1,),
        in_specs=[
            pl.BlockSpec((TS, N, D), lambda c: (jnp.minimum(c, last), 0, 0)),
            full((D, 3 * H)),
            full((H, 3 * H)),
            full((8, 3 * H)),
            full((8, 3 * H)),
            full((H, 3 * H)),
            full((H, 3 * H)),
            full((8, 3 * H)),
            full((8, 3 * H)),
            pl.BlockSpec((TS, N, LANE),
                         lambda c: (jnp.minimum(c, last), 0, 0)),
            pl.BlockSpec((TS, N, LANE),
                         lambda c: (jnp.maximum(c - 1, 0), 0, 0)),
            full((2, N, H)),
        ],
        out_specs=[
            pl.BlockSpec((TS, N, H), lambda c: (jnp.maximum(c - 1, 0), 0, 0)),
            full((2, N, H)),
        ],
        out_shape=[
            jax.ShapeDtypeStruct((T, N, H), jnp.float32),
            jax.ShapeDtypeStruct((2, N, H), jnp.float32),
        ],
        scratch_shapes=[
            pltpu.VMEM((N, H), jnp.float32),
            pltpu.VMEM((N, H), jnp.float32),
            pltpu.VMEM((TS * N, 3 * H), jnp.float32),
            pltpu.VMEM((TS * N, H), jnp.bfloat16),
            pltpu.VMEM((TS * N, 3 * H), jnp.float32),
        ],
        compiler_params=pltpu.CompilerParams(
            dimension_semantics=("arbitrary",),
        ),
    )(xs, W_ih0.T.astype(jnp.bfloat16), W_hh0.T.astype(jnp.bfloat16),
      prep_b(b_ih0), prep_b(b_hh0),
      W_ih1.T.astype(jnp.bfloat16), W_hh1.T.astype(jnp.bfloat16),
      prep_b(b_ih1), prep_b(b_hh1),
      masks_b, masks_b, hidden_states)

    return out1.reshape(T * N, H), h_n


# speedup vs baseline: 7.4032x; 1.2193x over previous
import functools

import jax
import jax.numpy as jnp
from jax.experimental import pallas as pl
from jax.experimental.pallas import tpu as pltpu

T = 256
N = 16
D = 512
H = 512
TS = 32
LANE = 128


def _gru_gate(h, m, gh_unbiased, bhh, gi, h_dim):
    gh = gh_unbiased + bhh
    r = jax.nn.sigmoid(gi[:, :h_dim] + gh[:, :h_dim])
    z = jax.nn.sigmoid(gi[:, h_dim:2 * h_dim] + gh[:, h_dim:2 * h_dim])
    cand = jnp.tanh(gi[:, 2 * h_dim:] + r * gh[:, 2 * h_dim:])
    return (1.0 - z) * cand + z * (h * m)


def _gru2_kernel(x_ref, wih0_ref, whh0_ref, bih0_ref, bhh0_ref,
                 wih1_ref, whh1_ref, bih1_ref, bhh1_ref, m0_ref, m1_ref,
                 hinit_ref, out_ref, hn_ref,
                 h0_ref, h1_ref, gi0_ref, mid_ref, gi1_ref,
                 *, ts, n, h_dim, nchunks):
    c = pl.program_id(0)

    @pl.when(c == 0)
    def _():
        h0_ref[...] = hinit_ref[0]
        h1_ref[...] = hinit_ref[1]
        gi1_ref[...] = jnp.zeros_like(gi1_ref)

    @pl.when(c < nchunks)
    def _():
        xc = x_ref[...].reshape(ts * n, x_ref.shape[2])
        gi0_ref[...] = (
            jnp.dot(xc, wih0_ref[...], preferred_element_type=jnp.float32)
            + bih0_ref[0:1, :]
        )

    l0_on = c < nchunks
    l1_on = c > 0

    def step(i, carry):
        h0, h1 = carry
        m0 = m0_ref[pl.ds(i, 1), :, :].reshape(n, LANE)[:, 0:1]
        m1 = m1_ref[pl.ds(i, 1), :, :].reshape(n, LANE)[:, 0:1]
        gh0 = jnp.dot((h0 * m0).astype(jnp.bfloat16), whh0_ref[...],
                      preferred_element_type=jnp.float32)
        gh1 = jnp.dot((h1 * m1).astype(jnp.bfloat16), whh1_ref[...],
                      preferred_element_type=jnp.float32)
        h0n = _gru_gate(h0, m0, gh0, bhh0_ref[0:1, :],
                        gi0_ref[pl.ds(i * n, n), :], h_dim)
        h1n = _gru_gate(h1, m1, gh1, bhh1_ref[0:1, :],
                        gi1_ref[pl.ds(i * n, n), :], h_dim)
        mid_ref[pl.ds(i * n, n), :] = h0n.astype(jnp.bfloat16)
        out_ref[pl.ds(i, 1)] = h1n.reshape(1, n, h_dim)
        h0 = jnp.where(l0_on, h0n, h0)
        h1 = jnp.where(l1_on, h1n, h1)
        return (h0, h1)

    h0, h1 = jax.lax.fori_loop(0, ts, step, (h0_ref[...], h1_ref[...]), unroll=4)
    h0_ref[...] = h0
    h1_ref[...] = h1

    @pl.when(c == nchunks)
    def _():
        hn_ref[0] = h0_ref[...]
        hn_ref[1] = h1_ref[...]

    @pl.when(c < nchunks)
    def _():
        gi1_ref[...] = (
            jnp.dot(mid_ref[...], wih1_ref[...],
                    preferred_element_type=jnp.float32)
            + bih1_ref[0:1, :]
        )


def kernel(x, hidden_states, masks, W_ih0, W_hh0, b_ih0, b_hh0,
           W_ih1, W_hh1, b_ih1, b_hh1):
    xs = x.reshape(T, N, D).astype(jnp.bfloat16)
    masks_b = jnp.broadcast_to(
        masks.astype(jnp.float32).reshape(T, N, 1), (T, N, LANE))

    def prep_b(b):
        return jnp.broadcast_to(b.reshape(1, 3 * H), (8, 3 * H))

    nchunks = T // TS
    last = nchunks - 1
    body = functools.partial(_gru2_kernel, ts=TS, n=N, h_dim=H,
                             nchunks=nchunks)
    full = lambda shape: pl.BlockSpec(shape, lambda c: (0,) * len(shape))
    out1, h_n = pl.pallas_call(
        body,
        grid=(nchunks + 1,),
        in_specs=[
            pl.BlockSpec((TS, N, D), lambda c: (jnp.minimum(c, last), 0, 0)),
            full((D, 3 * H)),
            full((H, 3 * H)),
            full((8, 3 * H)),
            full((8, 3 * H)),
            full((H, 3 * H)),
            full((H, 3 * H)),
            full((8, 3 * H)),
            full((8, 3 * H)),
            pl.BlockSpec((TS, N, LANE),
                         lambda c: (jnp.minimum(c, last), 0, 0)),
            pl.BlockSpec((TS, N, LANE),
                         lambda c: (jnp.maximum(c - 1, 0), 0, 0)),
            full((2, N, H)),
        ],
        out_specs=[
            pl.BlockSpec((TS, N, H), lambda c: (jnp.maximum(c - 1, 0), 0, 0)),
            full((2, N, H)),
        ],
        out_shape=[
            jax.ShapeDtypeStruct((T, N, H), jnp.float32),
            jax.ShapeDtypeStruct((2, N, H), jnp.float32),
        ],
        scratch_shapes=[
            pltpu.VMEM((N, H), jnp.float32),
            pltpu.VMEM((N, H), jnp.float32),
            pltpu.VMEM((TS * N, 3 * H), jnp.float32),
            pltpu.VMEM((TS * N, H), jnp.bfloat16),
            pltpu.VMEM((TS * N, 3 * H), jnp.float32),
        ],
        compiler_params=pltpu.CompilerParams(
            dimension_semantics=("arbitrary",),
        ),
    )(xs, W_ih0.T.astype(jnp.bfloat16), W_hh0.T.astype(jnp.bfloat16),
      prep_b(b_ih0), prep_b(b_hh0),
      W_ih1.T.astype(jnp.bfloat16), W_hh1.T.astype(jnp.bfloat16),
      prep_b(b_ih1), prep_b(b_hh1),
      masks_b, masks_b, hidden_states)

    return out1.reshape(T * N, H), h_n


# speedup vs baseline: 7.6685x; 1.0358x over previous
import functools

import jax
import jax.numpy as jnp
from jax.experimental import pallas as pl
from jax.experimental.pallas import tpu as pltpu

T = 256
N = 16
D = 512
H = 512
TS = 32
LANE = 128
UNROLL = 8


def _gru_gate(h, m, gh_unbiased, bhh, gi, h_dim):
    gh = gh_unbiased + bhh
    r = jax.nn.sigmoid(gi[:, :h_dim] + gh[:, :h_dim])
    z = jax.nn.sigmoid(gi[:, h_dim:2 * h_dim] + gh[:, h_dim:2 * h_dim])
    cand = jnp.tanh(gi[:, 2 * h_dim:] + r * gh[:, 2 * h_dim:])
    return (1.0 - z) * cand + z * (h * m)


def _gru2_kernel(x_ref, wih0_ref, whh0_ref, bih0_ref, bhh0_ref,
                 wih1_ref, whh1_ref, bih1_ref, bhh1_ref, m0_ref, m1_ref,
                 hinit_ref, out_ref, hn_ref,
                 h0_ref, h1_ref, gi0_ref, mid_ref, gi1_ref,
                 *, ts, n, h_dim, nchunks):
    c = pl.program_id(0)

    @pl.when(c == 0)
    def _():
        h0_ref[...] = hinit_ref[0]
        h1_ref[...] = hinit_ref[1]
        gi1_ref[...] = jnp.zeros_like(gi1_ref)

    @pl.when(c < nchunks)
    def _():
        xc = x_ref[...].reshape(ts * n, x_ref.shape[2])
        gi0_ref[...] = (
            jnp.dot(xc, wih0_ref[...], preferred_element_type=jnp.float32)
            + bih0_ref[0:1, :]
        )

    def step(i, carry):
        h0, h1 = carry
        m0 = m0_ref[pl.ds(i, 1), :, :].reshape(n, LANE)[:, 0:1]
        m1 = m1_ref[pl.ds(i, 1), :, :].reshape(n, LANE)[:, 0:1]
        gh0 = jnp.dot((h0 * m0).astype(jnp.bfloat16), whh0_ref[...],
                      preferred_element_type=jnp.float32)
        gh1 = jnp.dot((h1 * m1).astype(jnp.bfloat16), whh1_ref[...],
                      preferred_element_type=jnp.float32)
        h0n = _gru_gate(h0, m0, gh0, bhh0_ref[0:1, :],
                        gi0_ref[pl.ds(i * n, n), :], h_dim)
        h1n = _gru_gate(h1, m1, gh1, bhh1_ref[0:1, :],
                        gi1_ref[pl.ds(i * n, n), :], h_dim)
        mid_ref[pl.ds(i * n, n), :] = h0n.astype(jnp.bfloat16)
        out_ref[pl.ds(i, 1)] = h1n.reshape(1, n, h_dim)
        return (h0n, h1n)

    h0_prev = h0_ref[...]
    h1_prev = h1_ref[...]
    h0, h1 = jax.lax.fori_loop(0, ts, step, (h0_prev, h1_prev),
                               unroll=UNROLL)

    @pl.when(c == nchunks - 1)
    def _():
        hn_ref[0] = h0

    h0_ref[...] = jnp.where(c < nchunks, h0, h0_prev)
    h1_ref[...] = jnp.where(c > 0, h1, h1_prev)

    @pl.when(c == nchunks)
    def _():
        hn_ref[1] = h1_ref[...]

    @pl.when(c < nchunks)
    def _():
        gi1_ref[...] = (
            jnp.dot(mid_ref[...], wih1_ref[...],
                    preferred_element_type=jnp.float32)
            + bih1_ref[0:1, :]
        )


def kernel(x, hidden_states, masks, W_ih0, W_hh0, b_ih0, b_hh0,
           W_ih1, W_hh1, b_ih1, b_hh1):
    xs = x.reshape(T, N, D).astype(jnp.bfloat16)
    masks_b = jnp.broadcast_to(
        masks.astype(jnp.float32).reshape(T, N, 1), (T, N, LANE))

    def prep_b(b):
        return jnp.broadcast_to(b.reshape(1, 3 * H), (8, 3 * H))

    nchunks = T // TS
    last = nchunks - 1
    body = functools.partial(_gru2_kernel, ts=TS, n=N, h_dim=H,
                             nchunks=nchunks)
    full = lambda shape: pl.BlockSpec(shape, lambda c: (0,) * len(shape))
    out1, h_n = pl.pallas_call(
        body,
        grid=(nchunks + 1,),
        in_specs=[
            pl.BlockSpec((TS, N, D), lambda c: (jnp.minimum(c, last), 0, 0)),
            full((D, 3 * H)),
            full((H, 3 * H)),
            full((8, 3 * H)),
            full((8, 3 * H)),
            full((H, 3 * H)),
            full((H, 3 * H)),
            full((8, 3 * H)),
            full((8, 3 * H)),
            pl.BlockSpec((TS, N, LANE),
                         lambda c: (jnp.minimum(c, last), 0, 0)),
            pl.BlockSpec((TS, N, LANE),
                         lambda c: (jnp.maximum(c - 1, 0), 0, 0)),
            full((2, N, H)),
        ],
        out_specs=[
            pl.BlockSpec((TS, N, H), lambda c: (jnp.maximum(c - 1, 0), 0, 0)),
            full((2, N, H)),
        ],
        out_shape=[
            jax.ShapeDtypeStruct((T, N, H), jnp.float32),
            jax.ShapeDtypeStruct((2, N, H), jnp.float32),
        ],
        scratch_shapes=[
            pltpu.VMEM((N, H), jnp.float32),
            pltpu.VMEM((N, H), jnp.float32),
            pltpu.VMEM((TS * N, 3 * H), jnp.float32),
            pltpu.VMEM((TS * N, H), jnp.bfloat16),
            pltpu.VMEM((TS * N, 3 * H), jnp.float32),
        ],
        compiler_params=pltpu.CompilerParams(
            dimension_semantics=("arbitrary",),
        ),
    )(xs, W_ih0.T.astype(jnp.bfloat16), W_hh0.T.astype(jnp.bfloat16),
      prep_b(b_ih0), prep_b(b_hh0),
      W_ih1.T.astype(jnp.bfloat16), W_hh1.T.astype(jnp.bfloat16),
      prep_b(b_ih1), prep_b(b_hh1),
      masks_b, masks_b, hidden_states)

    return out1.reshape(T * N, H), h_n


# speedup vs baseline: 7.8330x; 1.0214x over previous
import functools

import jax
import jax.numpy as jnp
from jax.experimental import pallas as pl
from jax.experimental.pallas import tpu as pltpu

T = 256
N = 16
D = 512
H = 512
TS = 32
LANE = 128
UNROLL = 16


def _gru_gate(h, m, gh_unbiased, bhh, gi, h_dim):
    gh = gh_unbiased + bhh
    r = jax.nn.sigmoid(gi[:, :h_dim] + gh[:, :h_dim])
    z = jax.nn.sigmoid(gi[:, h_dim:2 * h_dim] + gh[:, h_dim:2 * h_dim])
    cand = jnp.tanh(gi[:, 2 * h_dim:] + r * gh[:, 2 * h_dim:])
    return (1.0 - z) * cand + z * (h * m)


def _gru2_kernel(x_ref, wih0_ref, whh0_ref, bih0_ref, bhh0_ref,
                 wih1_ref, whh1_ref, bih1_ref, bhh1_ref, m0_ref, m1_ref,
                 hinit_ref, out_ref, hn_ref,
                 h0_ref, h1_ref, gi0_ref, mid_ref, gi1_ref,
                 *, ts, n, h_dim, nchunks):
    c = pl.program_id(0)

    @pl.when(c == 0)
    def _():
        h0_ref[...] = hinit_ref[0]
        h1_ref[...] = hinit_ref[1]
        gi1_ref[...] = jnp.zeros_like(gi1_ref)

    @pl.when(c < nchunks)
    def _():
        xc = x_ref[...].reshape(ts * n, x_ref.shape[2])
        gi0_ref[...] = (
            jnp.dot(xc, wih0_ref[...], preferred_element_type=jnp.float32)
            + bih0_ref[0:1, :]
        )

    def step(i, carry):
        h0, h1 = carry
        m0 = m0_ref[pl.ds(i, 1), :, :].reshape(n, LANE)[:, 0:1]
        m1 = m1_ref[pl.ds(i, 1), :, :].reshape(n, LANE)[:, 0:1]
        gh0 = jnp.dot((h0 * m0).astype(jnp.bfloat16), whh0_ref[...],
                      preferred_element_type=jnp.float32)
        gh1 = jnp.dot((h1 * m1).astype(jnp.bfloat16), whh1_ref[...],
                      preferred_element_type=jnp.float32)
        h0n = _gru_gate(h0, m0, gh0, bhh0_ref[0:1, :],
                        gi0_ref[pl.ds(i * n, n), :], h_dim)
        h1n = _gru_gate(h1, m1, gh1, bhh1_ref[0:1, :],
                        gi1_ref[pl.ds(i * n, n), :], h_dim)
        mid_ref[pl.ds(i * n, n), :] = h0n.astype(jnp.bfloat16)
        out_ref[pl.ds(i, 1)] = h1n.reshape(1, n, h_dim)
        return (h0n, h1n)

    h0_prev = h0_ref[...]
    h1_prev = h1_ref[...]
    h0, h1 = jax.lax.fori_loop(0, ts, step, (h0_prev, h1_prev),
                               unroll=UNROLL)

    @pl.when(c == nchunks - 1)
    def _():
        hn_ref[0] = h0

    h0_ref[...] = jnp.where(c < nchunks, h0, h0_prev)
    h1_ref[...] = jnp.where(c > 0, h1, h1_prev)

    @pl.when(c == nchunks)
    def _():
        hn_ref[1] = h1_ref[...]

    @pl.when(c < nchunks)
    def _():
        gi1_ref[...] = (
            jnp.dot(mid_ref[...], wih1_ref[...],
                    preferred_element_type=jnp.float32)
            + bih1_ref[0:1, :]
        )


def kernel(x, hidden_states, masks, W_ih0, W_hh0, b_ih0, b_hh0,
           W_ih1, W_hh1, b_ih1, b_hh1):
    xs = x.reshape(T, N, D).astype(jnp.bfloat16)
    masks_b = jnp.broadcast_to(
        masks.astype(jnp.float32).reshape(T, N, 1), (T, N, LANE))

    def prep_b(b):
        return jnp.broadcast_to(b.reshape(1, 3 * H), (8, 3 * H))

    nchunks = T // TS
    last = nchunks - 1
    body = functools.partial(_gru2_kernel, ts=TS, n=N, h_dim=H,
                             nchunks=nchunks)
    full = lambda shape: pl.BlockSpec(shape, lambda c: (0,) * len(shape))
    out1, h_n = pl.pallas_call(
        body,
        grid=(nchunks + 1,),
        in_specs=[
            pl.BlockSpec((TS, N, D), lambda c: (jnp.minimum(c, last), 0, 0)),
            full((D, 3 * H)),
            full((H, 3 * H)),
            full((8, 3 * H)),
            full((8, 3 * H)),
            full((H, 3 * H)),
            full((H, 3 * H)),
            full((8, 3 * H)),
            full((8, 3 * H)),
            pl.BlockSpec((TS, N, LANE),
                         lambda c: (jnp.minimum(c, last), 0, 0)),
            pl.BlockSpec((TS, N, LANE),
                         lambda c: (jnp.maximum(c - 1, 0), 0, 0)),
            full((2, N, H)),
        ],
        out_specs=[
            pl.BlockSpec((TS, N, H), lambda c: (jnp.maximum(c - 1, 0), 0, 0)),
            full((2, N, H)),
        ],
        out_shape=[
            jax.ShapeDtypeStruct((T, N, H), jnp.float32),
            jax.ShapeDtypeStruct((2, N, H), jnp.float32),
        ],
        scratch_shapes=[
            pltpu.VMEM((N, H), jnp.float32),
            pltpu.VMEM((N, H), jnp.float32),
            pltpu.VMEM((TS * N, 3 * H), jnp.float32),
            pltpu.VMEM((TS * N, H), jnp.bfloat16),
            pltpu.VMEM((TS * N, 3 * H), jnp.float32),
        ],
        compiler_params=pltpu.CompilerParams(
            dimension_semantics=("arbitrary",),
        ),
    )(xs, W_ih0.T.astype(jnp.bfloat16), W_hh0.T.astype(jnp.bfloat16),
      prep_b(b_ih0), prep_b(b_hh0),
      W_ih1.T.astype(jnp.bfloat16), W_hh1.T.astype(jnp.bfloat16),
      prep_b(b_ih1), prep_b(b_hh1),
      masks_b, masks_b, hidden_states)

    return out1.reshape(T * N, H), h_n


# speedup vs baseline: 7.9115x; 1.0100x over previous
import functools

import jax
import jax.numpy as jnp
from jax.experimental import pallas as pl
from jax.experimental.pallas import tpu as pltpu

T = 256
N = 16
D = 512
H = 512
TS = 32
LANE = 128
UNROLL = 32


def _gru_gate(h, m, gh_unbiased, bhh, gi, h_dim):
    gh = gh_unbiased + bhh
    r = jax.nn.sigmoid(gi[:, :h_dim] + gh[:, :h_dim])
    z = jax.nn.sigmoid(gi[:, h_dim:2 * h_dim] + gh[:, h_dim:2 * h_dim])
    cand = jnp.tanh(gi[:, 2 * h_dim:] + r * gh[:, 2 * h_dim:])
    return (1.0 - z) * cand + z * (h * m)


def _gru2_kernel(x_ref, wih0_ref, whh0_ref, bih0_ref, bhh0_ref,
                 wih1_ref, whh1_ref, bih1_ref, bhh1_ref, m0_ref, m1_ref,
                 hinit_ref, out_ref, hn_ref,
                 h0_ref, h1_ref, gi0_ref, mid_ref, gi1_ref,
                 *, ts, n, h_dim, nchunks):
    c = pl.program_id(0)

    @pl.when(c == 0)
    def _():
        h0_ref[...] = hinit_ref[0]
        h1_ref[...] = hinit_ref[1]
        gi1_ref[...] = jnp.zeros_like(gi1_ref)

    @pl.when(c < nchunks)
    def _():
        xc = x_ref[...].reshape(ts * n, x_ref.shape[2])
        gi0_ref[...] = (
            jnp.dot(xc, wih0_ref[...], preferred_element_type=jnp.float32)
            + bih0_ref[0:1, :]
        )

    def step(i, carry):
        h0, h1 = carry
        m0 = m0_ref[pl.ds(i, 1), :, :].reshape(n, LANE)[:, 0:1]
        m1 = m1_ref[pl.ds(i, 1), :, :].reshape(n, LANE)[:, 0:1]
        gh0 = jnp.dot((h0 * m0).astype(jnp.bfloat16), whh0_ref[...],
                      preferred_element_type=jnp.float32)
        gh1 = jnp.dot((h1 * m1).astype(jnp.bfloat16), whh1_ref[...],
                      preferred_element_type=jnp.float32)
        h0n = _gru_gate(h0, m0, gh0, bhh0_ref[0:1, :],
                        gi0_ref[pl.ds(i * n, n), :], h_dim)
        h1n = _gru_gate(h1, m1, gh1, bhh1_ref[0:1, :],
                        gi1_ref[pl.ds(i * n, n), :], h_dim)
        mid_ref[pl.ds(i * n, n), :] = h0n.astype(jnp.bfloat16)
        out_ref[pl.ds(i, 1)] = h1n.reshape(1, n, h_dim)
        return (h0n, h1n)

    h0_prev = h0_ref[...]
    h1_prev = h1_ref[...]
    h0, h1 = jax.lax.fori_loop(0, ts, step, (h0_prev, h1_prev),
                               unroll=UNROLL)

    @pl.when(c == nchunks - 1)
    def _():
        hn_ref[0] = h0

    h0_ref[...] = jnp.where(c < nchunks, h0, h0_prev)
    h1_ref[...] = jnp.where(c > 0, h1, h1_prev)

    @pl.when(c == nchunks)
    def _():
        hn_ref[1] = h1_ref[...]

    @pl.when(c < nchunks)
    def _():
        gi1_ref[...] = (
            jnp.dot(mid_ref[...], wih1_ref[...],
                    preferred_element_type=jnp.float32)
            + bih1_ref[0:1, :]
        )


def kernel(x, hidden_states, masks, W_ih0, W_hh0, b_ih0, b_hh0,
           W_ih1, W_hh1, b_ih1, b_hh1):
    xs = x.reshape(T, N, D).astype(jnp.bfloat16)
    masks_b = jnp.broadcast_to(
        masks.astype(jnp.float32).reshape(T, N, 1), (T, N, LANE))

    def prep_b(b):
        return jnp.broadcast_to(b.reshape(1, 3 * H), (8, 3 * H))

    nchunks = T // TS
    last = nchunks - 1
    body = functools.partial(_gru2_kernel, ts=TS, n=N, h_dim=H,
                             nchunks=nchunks)
    full = lambda shape: pl.BlockSpec(shape, lambda c: (0,) * len(shape))
    out1, h_n = pl.pallas_call(
        body,
        grid=(nchunks + 1,),
        in_specs=[
            pl.BlockSpec((TS, N, D), lambda c: (jnp.minimum(c, last), 0, 0)),
            full((D, 3 * H)),
            full((H, 3 * H)),
            full((8, 3 * H)),
            full((8, 3 * H)),
            full((H, 3 * H)),
            full((H, 3 * H)),
            full((8, 3 * H)),
            full((8, 3 * H)),
            pl.BlockSpec((TS, N, LANE),
                         lambda c: (jnp.minimum(c, last), 0, 0)),
            pl.BlockSpec((TS, N, LANE),
                         lambda c: (jnp.maximum(c - 1, 0), 0, 0)),
            full((2, N, H)),
        ],
        out_specs=[
            pl.BlockSpec((TS, N, H), lambda c: (jnp.maximum(c - 1, 0), 0, 0)),
            full((2, N, H)),
        ],
        out_shape=[
            jax.ShapeDtypeStruct((T, N, H), jnp.float32),
            jax.ShapeDtypeStruct((2, N, H), jnp.float32),
        ],
        scratch_shapes=[
            pltpu.VMEM((N, H), jnp.float32),
            pltpu.VMEM((N, H), jnp.float32),
            pltpu.VMEM((TS * N, 3 * H), jnp.float32),
            pltpu.VMEM((TS * N, H), jnp.bfloat16),
            pltpu.VMEM((TS * N, 3 * H), jnp.float32),
        ],
        compiler_params=pltpu.CompilerParams(
            dimension_semantics=("arbitrary",),
        ),
    )(xs, W_ih0.T.astype(jnp.bfloat16), W_hh0.T.astype(jnp.bfloat16),
      prep_b(b_ih0), prep_b(b_hh0),
      W_ih1.T.astype(jnp.bfloat16), W_hh1.T.astype(jnp.bfloat16),
      prep_b(b_ih1), prep_b(b_hh1),
      masks_b, masks_b, hidden_states)

    return out1.reshape(T * N, H), h_n


# speedup vs baseline: 7.9150x; 1.0004x over previous
import functools

import jax
import jax.numpy as jnp
from jax.experimental import pallas as pl
from jax.experimental.pallas import tpu as pltpu

T = 256
N = 16
D = 512
H = 512
TS = 32
LANE = 128
UNROLL = 32


def _gru_gate(h, m, gh_unbiased, bhh, gi_b, h_dim):
    gi = gi_b.astype(jnp.float32)
    gh = gh_unbiased + bhh
    r = jax.nn.sigmoid(gi[:, :h_dim] + gh[:, :h_dim])
    z = jax.nn.sigmoid(gi[:, h_dim:2 * h_dim] + gh[:, h_dim:2 * h_dim])
    cand = jnp.tanh(gi[:, 2 * h_dim:] + r * gh[:, 2 * h_dim:])
    return (1.0 - z) * cand + z * (h * m)


def _gru2_kernel(x_ref, wih0_ref, whh0_ref, bih0_ref, bhh0_ref,
                 wih1_ref, whh1_ref, bih1_ref, bhh1_ref, m0_ref, m1_ref,
                 hinit_ref, out_ref, hn_ref,
                 h0_ref, h1_ref, gi0_ref, mid_ref, gi1_ref,
                 *, ts, n, h_dim, nchunks):
    c = pl.program_id(0)

    @pl.when(c == 0)
    def _():
        h0_ref[...] = hinit_ref[0]
        h1_ref[...] = hinit_ref[1]
        gi1_ref[...] = jnp.zeros_like(gi1_ref)

    @pl.when(c < nchunks)
    def _():
        xc = x_ref[...].reshape(ts * n, x_ref.shape[2])
        gi0_ref[...] = (
            jnp.dot(xc, wih0_ref[...], preferred_element_type=jnp.float32)
            + bih0_ref[0:1, :]
        ).astype(jnp.bfloat16)

    def step(i, carry):
        h0, h1 = carry
        m0 = m0_ref[pl.ds(i, 1), :, :].reshape(n, LANE)[:, 0:1]
        m1 = m1_ref[pl.ds(i, 1), :, :].reshape(n, LANE)[:, 0:1]
        gh0 = jnp.dot((h0 * m0).astype(jnp.bfloat16), whh0_ref[...],
                      preferred_element_type=jnp.float32)
        gh1 = jnp.dot((h1 * m1).astype(jnp.bfloat16), whh1_ref[...],
                      preferred_element_type=jnp.float32)
        h0n = _gru_gate(h0, m0, gh0, bhh0_ref[0:1, :],
                        gi0_ref[pl.ds(i * n, n), :], h_dim)
        h1n = _gru_gate(h1, m1, gh1, bhh1_ref[0:1, :],
                        gi1_ref[pl.ds(i * n, n), :], h_dim)
        mid_ref[pl.ds(i * n, n), :] = h0n.astype(jnp.bfloat16)
        out_ref[pl.ds(i, 1)] = h1n.reshape(1, n, h_dim)
        return (h0n, h1n)

    h0_prev = h0_ref[...]
    h1_prev = h1_ref[...]
    h0, h1 = jax.lax.fori_loop(0, ts, step, (h0_prev, h1_prev),
                               unroll=UNROLL)

    @pl.when(c == nchunks - 1)
    def _():
        hn_ref[0] = h0

    h0_ref[...] = jnp.where(c < nchunks, h0, h0_prev)
    h1_ref[...] = jnp.where(c > 0, h1, h1_prev)

    @pl.when(c == nchunks)
    def _():
        hn_ref[1] = h1_ref[...]

    @pl.when(c < nchunks)
    def _():
        gi1_ref[...] = (
            jnp.dot(mid_ref[...], wih1_ref[...],
                    preferred_element_type=jnp.float32)
            + bih1_ref[0:1, :]
        ).astype(jnp.bfloat16)


def kernel(x, hidden_states, masks, W_ih0, W_hh0, b_ih0, b_hh0,
           W_ih1, W_hh1, b_ih1, b_hh1):
    xs = x.reshape(T, N, D).astype(jnp.bfloat16)
    masks_b = jnp.broadcast_to(
        masks.astype(jnp.float32).reshape(T, N, 1), (T, N, LANE))

    def prep_b(b):
        return jnp.broadcast_to(b.reshape(1, 3 * H), (8, 3 * H))

    nchunks = T // TS
    last = nchunks - 1
    body = functools.partial(_gru2_kernel, ts=TS, n=N, h_dim=H,
                             nchunks=nchunks)
    full = lambda shape: pl.BlockSpec(shape, lambda c: (0,) * len(shape))
    out1, h_n = pl.pallas_call(
        body,
        grid=(nchunks + 1,),
        in_specs=[
            pl.BlockSpec((TS, N, D), lambda c: (jnp.minimum(c, last), 0, 0)),
            full((D, 3 * H)),
            full((H, 3 * H)),
            full((8, 3 * H)),
            full((8, 3 * H)),
            full((H, 3 * H)),
            full((H, 3 * H)),
            full((8, 3 * H)),
            full((8, 3 * H)),
            pl.BlockSpec((TS, N, LANE),
                         lambda c: (jnp.minimum(c, last), 0, 0)),
            pl.BlockSpec((TS, N, LANE),
                         lambda c: (jnp.maximum(c - 1, 0), 0, 0)),
            full((2, N, H)),
        ],
        out_specs=[
            pl.BlockSpec((TS, N, H), lambda c: (jnp.maximum(c - 1, 0), 0, 0)),
            full((2, N, H)),
        ],
        out_shape=[
            jax.ShapeDtypeStruct((T, N, H), jnp.float32),
            jax.ShapeDtypeStruct((2, N, H), jnp.float32),
        ],
        scratch_shapes=[
            pltpu.VMEM((N, H), jnp.float32),
            pltpu.VMEM((N, H), jnp.float32),
            pltpu.VMEM((TS * N, 3 * H), jnp.bfloat16),
            pltpu.VMEM((TS * N, H), jnp.bfloat16),
            pltpu.VMEM((TS * N, 3 * H), jnp.bfloat16),
        ],
        compiler_params=pltpu.CompilerParams(
            dimension_semantics=("arbitrary",),
        ),
    )(xs, W_ih0.T.astype(jnp.bfloat16), W_hh0.T.astype(jnp.bfloat16),
      prep_b(b_ih0), prep_b(b_hh0),
      W_ih1.T.astype(jnp.bfloat16), W_hh1.T.astype(jnp.bfloat16),
      prep_b(b_ih1), prep_b(b_hh1),
      masks_b, masks_b, hidden_states)

    return out1.reshape(T * N, H), h_n


# speedup vs baseline: 8.2823x; 1.0464x over previous
import functools

import jax
import jax.numpy as jnp
from jax.experimental import pallas as pl
from jax.experimental.pallas import tpu as pltpu

T = 256
N = 16
D = 512
H = 512
TS = 32
LANE = 128
UNROLL = 32


def _gru_gate(h, m, gh_unbiased, bhh, gi_b, h_dim):
    gi = gi_b.astype(jnp.float32)
    gh = gh_unbiased + bhh
    r = jax.nn.sigmoid(gi[:, :h_dim] + gh[:, :h_dim])
    z = jax.nn.sigmoid(gi[:, h_dim:2 * h_dim] + gh[:, h_dim:2 * h_dim])
    cand = jnp.tanh(gi[:, 2 * h_dim:] + r * gh[:, 2 * h_dim:])
    return (1.0 - z) * cand + z * (h * m)


def _gru2_kernel(x_ref, wih0_ref, whh0_ref, bih0_ref, bhh0_ref,
                 wih1_ref, whh1_ref, bih1_ref, bhh1_ref, m0_ref, m1_ref,
                 hinit_ref, out_ref, hn_ref,
                 h0_ref, h1_ref, gi0_ref, mid_ref, gi1_ref,
                 *, ts, n, h_dim, nchunks):
    c = pl.program_id(0)

    @pl.when(c == 0)
    def _():
        h0_ref[...] = hinit_ref[0]
        h1_ref[...] = hinit_ref[1]

    @pl.when(c < nchunks)
    def _():
        xc = x_ref[...].reshape(ts * n, x_ref.shape[2])
        gi0_ref[...] = (
            jnp.dot(xc, wih0_ref[...], preferred_element_type=jnp.float32)
            + bih0_ref[0:1, :]
        ).astype(jnp.bfloat16)

    def l0_step(i, h0):
        m0 = m0_ref[pl.ds(i, 1), :, :].reshape(n, LANE)[:, 0:1]
        gh0 = jnp.dot((h0 * m0).astype(jnp.bfloat16), whh0_ref[...],
                      preferred_element_type=jnp.float32)
        h0n = _gru_gate(h0, m0, gh0, bhh0_ref[0:1, :],
                        gi0_ref[pl.ds(i * n, n), :], h_dim)
        mid_ref[pl.ds(i * n, n), :] = h0n.astype(jnp.bfloat16)
        return h0n

    def l1_step(i, h1):
        m1 = m1_ref[pl.ds(i, 1), :, :].reshape(n, LANE)[:, 0:1]
        gh1 = jnp.dot((h1 * m1).astype(jnp.bfloat16), whh1_ref[...],
                      preferred_element_type=jnp.float32)
        h1n = _gru_gate(h1, m1, gh1, bhh1_ref[0:1, :],
                        gi1_ref[pl.ds(i * n, n), :], h_dim)
        out_ref[pl.ds(i, 1)] = h1n.reshape(1, n, h_dim)
        return h1n

    def joint_step(i, carry):
        h0, h1 = carry
        return (l0_step(i, h0), l1_step(i, h1))

    @pl.when(c == 0)
    def _():
        h0_ref[...] = jax.lax.fori_loop(0, ts, l0_step, h0_ref[...],
                                        unroll=UNROLL)

    @pl.when(jnp.logical_and(c > 0, c < nchunks))
    def _():
        h0, h1 = jax.lax.fori_loop(0, ts, joint_step,
                                   (h0_ref[...], h1_ref[...]),
                                   unroll=UNROLL)
        h0_ref[...] = h0
        h1_ref[...] = h1

    @pl.when(c == nchunks)
    def _():
        h1_ref[...] = jax.lax.fori_loop(0, ts, l1_step, h1_ref[...],
                                        unroll=UNROLL)

    @pl.when(c == nchunks - 1)
    def _():
        hn_ref[0] = h0_ref[...]

    @pl.when(c == nchunks)
    def _():
        hn_ref[1] = h1_ref[...]

    @pl.when(c < nchunks)
    def _():
        gi1_ref[...] = (
            jnp.dot(mid_ref[...], wih1_ref[...],
                    preferred_element_type=jnp.float32)
            + bih1_ref[0:1, :]
        ).astype(jnp.bfloat16)


def kernel(x, hidden_states, masks, W_ih0, W_hh0, b_ih0, b_hh0,
           W_ih1, W_hh1, b_ih1, b_hh1):
    xs = x.reshape(T, N, D).astype(jnp.bfloat16)
    masks_b = jnp.broadcast_to(
        masks.astype(jnp.float32).reshape(T, N, 1), (T, N, LANE))

    def prep_b(b):
        return jnp.broadcast_to(b.reshape(1, 3 * H), (8, 3 * H))

    nchunks = T // TS
    last = nchunks - 1
    body = functools.partial(_gru2_kernel, ts=TS, n=N, h_dim=H,
                             nchunks=nchunks)
    full = lambda shape: pl.BlockSpec(shape, lambda c: (0,) * len(shape))
    out1, h_n = pl.pallas_call(
        body,
        grid=(nchunks + 1,),
        in_specs=[
            pl.BlockSpec((TS, N, D), lambda c: (jnp.minimum(c, last), 0, 0)),
            full((D, 3 * H)),
            full((H, 3 * H)),
            full((8, 3 * H)),
            full((8, 3 * H)),
            full((H, 3 * H)),
            full((H, 3 * H)),
            full((8, 3 * H)),
            full((8, 3 * H)),
            pl.BlockSpec((TS, N, LANE),
                         lambda c: (jnp.minimum(c, last), 0, 0)),
            pl.BlockSpec((TS, N, LANE),
                         lambda c: (jnp.maximum(c - 1, 0), 0, 0)),
            full((2, N, H)),
        ],
        out_specs=[
            pl.BlockSpec((TS, N, H), lambda c: (jnp.maximum(c - 1, 0), 0, 0)),
            full((2, N, H)),
        ],
        out_shape=[
            jax.ShapeDtypeStruct((T, N, H), jnp.float32),
            jax.ShapeDtypeStruct((2, N, H), jnp.float32),
        ],
        scratch_shapes=[
            pltpu.VMEM((N, H), jnp.float32),
            pltpu.VMEM((N, H), jnp.float32),
            pltpu.VMEM((TS * N, 3 * H), jnp.bfloat16),
            pltpu.VMEM((TS * N, H), jnp.bfloat16),
            pltpu.VMEM((TS * N, 3 * H), jnp.bfloat16),
        ],
        compiler_params=pltpu.CompilerParams(
            dimension_semantics=("arbitrary",),
        ),
    )(xs, W_ih0.T.astype(jnp.bfloat16), W_hh0.T.astype(jnp.bfloat16),
      prep_b(b_ih0), prep_b(b_hh0),
      W_ih1.T.astype(jnp.bfloat16), W_hh1.T.astype(jnp.bfloat16),
      prep_b(b_ih1), prep_b(b_hh1),
      masks_b, masks_b, hidden_states)

    return out1.reshape(T * N, H), h_n
